# Initial kernel scaffold; baseline (speedup 1.0000x reference)
import jax
import jax.numpy as jnp
from jax.experimental import pallas as pl


def kernel(x):
    raise NotImplementedError("write your pallas kernel here")


# speedup vs baseline: 3.1077x; 3.1077x over previous
import ml_dtypes
import numpy as np
import jax
import jax.numpy as jnp
from jax.experimental import pallas as pl

_C = 32
_L = 500
_NF = _L // 2 + 1
_NS = _L // 2 - 1
_NPK = 3
_WIN = 11
_SPB = 8
_ROWS = _SPB * _C
_NPAD = 256

_OMEGA = float(np.float32(-2.0 * np.pi / _L))

_k3 = np.arange(-1, 2, dtype=np.float32)
_gk = np.exp(-_k3 ** 2 / (2.0 * np.float32(2.0) ** 2)).astype(np.float32)
_gk = (_gk / _gk.sum()).astype(np.float32)
_GA16 = float(np.float32(_gk[0].astype(ml_dtypes.bfloat16)))
_GB16 = float(np.float32(_gk[1].astype(ml_dtypes.bfloat16)))

_FREQS = (np.arange(_NF, dtype=np.float32)
          * np.float32(1.0 / (_L * (1.0 / 128.0))))[1:_NF - 1]
_FREQ_ROW = np.zeros((_SPB, _NPAD), dtype=np.float32)
_FREQ_ROW[:, :_NS] = _FREQS[None, :]


def _twiddle_body(c_ref, s_ref):
    j = jax.lax.broadcasted_iota(jnp.uint32, (_L, _NPAD), 0)
    k = jax.lax.broadcasted_iota(jnp.uint32, (_L, _NPAD), 1)
    theta = ((j * k) % jnp.uint32(_L)).astype(jnp.float32) * jnp.float32(_OMEGA)
    c = jnp.cos(theta)
    s = jnp.sin(theta)
    c_ref[...] = c
    s_ref[...] = s - c


def _shift_right(a, s, fill):
    pad = jnp.full(a.shape[:-1] + (s,), fill, dtype=a.dtype)
    return jnp.concatenate([pad, a[..., :-s]], axis=-1)


def _shift_left(a, s, fill):
    pad = jnp.full(a.shape[:-1] + (s,), fill, dtype=a.dtype)
    return jnp.concatenate([a[..., s:], pad], axis=-1)


def _body(x_ref, c_ref, s_ref, rf_ref, fq_ref, fd_ref, anch_ref):
    xb = x_ref[...]
    re = jax.lax.dot_general(
        xb, c_ref[...], (((1,), (0,)), ((), ())),
        preferred_element_type=jnp.float32,
        precision=jax.lax.Precision.HIGHEST)
    bm = jax.lax.dot_general(
        xb, s_ref[...], (((1,), (0,)), ((), ())),
        preferred_element_type=jnp.float32,
        precision=jax.lax.Precision.HIGHEST)
    im = re + bm
    fd = re * re + im * im
    fd_ref[...] = fd[:, :_NF]

    fd3 = fd.reshape(_SPB, _C, _NPAD)
    acc = fd3[:, 0, :]
    for c in range(1, _C):
        acc = acc + fd3[:, c, :]
    fm = acc * jnp.float32(0.03125)
    f = fm[:, 1:1 + _NS]

    fb = f.astype(jnp.bfloat16).astype(jnp.float32)
    fl = _shift_right(fb, 1, 0.0)
    fr = _shift_left(fb, 1, 0.0)
    sm = _GA16 * fl + _GB16 * fb + _GA16 * fr

    inf = jnp.float32(jnp.inf)
    sm_l = _shift_right(sm, 1, inf)
    sm_r = _shift_left(sm, 1, inf)
    peak = ((sm - sm_l) > 0) & ((sm_r - sm) < 0)

    lmax = jnp.full(sm.shape, -1.0, dtype=sm.dtype)
    rmax = jnp.full(sm.shape, -1.0, dtype=sm.dtype)
    for s in range(1, _WIN + 1):
        lmax = jnp.maximum(lmax, _shift_right(sm, s, -1.0))
        rmax = jnp.maximum(rmax, _shift_left(sm, s, -1.0))
    valid = peak & (sm > lmax) & (sm >= rmax)

    score = jnp.where(valid, sm, -1.0)
    iota = jax.lax.broadcasted_iota(jnp.int32, (_SPB, _NS), 1)
    freq_row = fq_ref[...][:, :_NS]

    pfs = []
    for j in range(_NPK):
        m = jnp.max(score, axis=1)
        idx = jnp.argmax(score, axis=1)
        hit = iota == idx[:, None]
        freq = jnp.sum(jnp.where(hit, freq_row, 0.0), axis=1)
        exists = m >= 0.0
        pfs.append(jnp.where(exists, freq, rf_ref[:, j]))
        score = jnp.where(hit, -1.0, score)

    p0, p1, p2 = pfs
    ab_lo = jnp.minimum(p0, p1)
    ab_hi = jnp.maximum(p0, p1)
    lo = jnp.minimum(ab_lo, p2)
    hi = jnp.maximum(ab_hi, p2)
    mid = jnp.maximum(ab_lo, jnp.minimum(ab_hi, p2))

    c1 = (mid - lo) / 2.0 + lo
    c2 = (hi - mid) / 2.0 + mid

    col = jax.lax.broadcasted_iota(jnp.int32, (_SPB, 128), 1)
    out = jnp.where(col == 0, jnp.float32(0.5),
          jnp.where(col <= 2, c1[:, None],
          jnp.where(col <= 4, c2[:, None],
          jnp.where(col == 5, jnp.float32(45.0), jnp.float32(0.0)))))
    anch_ref[...] = out


def kernel(x):
    x2 = x.reshape(-1, _L)
    r = x2.shape[0]
    b = r // _C

    cmat, smat = pl.pallas_call(
        _twiddle_body,
        out_specs=[pl.BlockSpec((_L, _NPAD), lambda: (0, 0)),
                   pl.BlockSpec((_L, _NPAD), lambda: (0, 0))],
        out_shape=[jax.ShapeDtypeStruct((_L, _NPAD), jnp.float32),
                   jax.ShapeDtypeStruct((_L, _NPAD), jnp.float32)],
    )()

    rkey = jax.random.fold_in(jax.random.key(8989), 1)
    rand_idx = jax.random.randint(rkey, (b, _NPK), 0, _NS)
    rand_freq = jnp.take(jnp.asarray(_FREQS), rand_idx, axis=0)
    rf = jnp.zeros((b, 128), dtype=jnp.float32).at[:, :_NPK].set(rand_freq)

    grid = (r // _ROWS,)
    fd, anch = pl.pallas_call(
        _body,
        grid=grid,
        in_specs=[
            pl.BlockSpec((_ROWS, _L), lambda i: (i, 0)),
            pl.BlockSpec((_L, _NPAD), lambda i: (0, 0)),
            pl.BlockSpec((_L, _NPAD), lambda i: (0, 0)),
            pl.BlockSpec((_SPB, 128), lambda i: (i, 0)),
            pl.BlockSpec((_SPB, _NPAD), lambda i: (0, 0)),
        ],
        out_specs=[
            pl.BlockSpec((_ROWS, _NF), lambda i: (i, 0)),
            pl.BlockSpec((_SPB, 128), lambda i: (i, 0)),
        ],
        out_shape=[
            jax.ShapeDtypeStruct((r, _NF), jnp.float32),
            jax.ShapeDtypeStruct((b, 128), jnp.float32),
        ],
    )(x2, cmat, smat, rf, jnp.asarray(_FREQ_ROW))

    anchors = anch[:, :6].reshape(b, 3, 2)
    f_ = fd.reshape(b, _C, _NF)
    return (anchors, f_)


# speedup vs baseline: 3.7984x; 1.2223x over previous
import ml_dtypes
import numpy as np
import jax
import jax.numpy as jnp
from jax.experimental import pallas as pl

_C = 32
_L = 500
_NF = _L // 2 + 1
_NS = _L // 2 - 1
_NPK = 3
_WIN = 11
_SPB = 16
_ROWS = _SPB * _C
_NPAD = 256

_OMEGA = float(np.float32(-2.0 * np.pi / _L))

_k3 = np.arange(-1, 2, dtype=np.float32)
_gk = np.exp(-_k3 ** 2 / (2.0 * np.float32(2.0) ** 2)).astype(np.float32)
_gk = (_gk / _gk.sum()).astype(np.float32)
_GA16 = float(np.float32(_gk[0].astype(ml_dtypes.bfloat16)))
_GB16 = float(np.float32(_gk[1].astype(ml_dtypes.bfloat16)))

_FREQS = (np.arange(_NF, dtype=np.float32)
          * np.float32(1.0 / (_L * (1.0 / 128.0))))[1:_NF - 1]
_FREQ_ROW = np.zeros((_SPB, _NPAD), dtype=np.float32)
_FREQ_ROW[:, :_NS] = _FREQS[None, :]


def _twiddle_body(c_ref, s_ref):
    j = jax.lax.broadcasted_iota(jnp.uint32, (_L, _NPAD), 0)
    k = jax.lax.broadcasted_iota(jnp.uint32, (_L, _NPAD), 1)
    theta = ((j * k) % jnp.uint32(_L)).astype(jnp.float32) * jnp.float32(_OMEGA)
    c = jnp.cos(theta)
    s = jnp.sin(theta)
    c_ref[...] = c
    s_ref[...] = s - c


def _shift_right(a, s, fill):
    pad = jnp.full(a.shape[:-1] + (s,), fill, dtype=a.dtype)
    return jnp.concatenate([pad, a[..., :-s]], axis=-1)


def _shift_left(a, s, fill):
    pad = jnp.full(a.shape[:-1] + (s,), fill, dtype=a.dtype)
    return jnp.concatenate([a[..., s:], pad], axis=-1)


def _body(x_ref, c_ref, s_ref, rf_ref, fq_ref, fd_ref, anch_ref):
    xb = x_ref[...]
    re = jax.lax.dot_general(
        xb, c_ref[...], (((1,), (0,)), ((), ())),
        preferred_element_type=jnp.float32,
        precision=jax.lax.Precision.HIGHEST)
    bm = jax.lax.dot_general(
        xb, s_ref[...], (((1,), (0,)), ((), ())),
        preferred_element_type=jnp.float32,
        precision=jax.lax.Precision.HIGHEST)
    im = re + bm
    fd = re * re + im * im
    fd_ref[...] = fd[:, :_NF]

    fd3 = fd.reshape(_SPB, _C, _NPAD)
    acc = fd3[:, 0, :]
    for c in range(1, _C):
        acc = acc + fd3[:, c, :]
    fm = acc * jnp.float32(0.03125)
    f = fm[:, 1:1 + _NS]

    fb = f.astype(jnp.bfloat16).astype(jnp.float32)
    fl = _shift_right(fb, 1, 0.0)
    fr = _shift_left(fb, 1, 0.0)
    sm = _GA16 * fl + _GB16 * fb + _GA16 * fr

    inf = jnp.float32(jnp.inf)
    sm_l = _shift_right(sm, 1, inf)
    sm_r = _shift_left(sm, 1, inf)
    peak = ((sm - sm_l) > 0) & ((sm_r - sm) < 0)

    lmax = jnp.full(sm.shape, -1.0, dtype=sm.dtype)
    rmax = jnp.full(sm.shape, -1.0, dtype=sm.dtype)
    for s in range(1, _WIN + 1):
        lmax = jnp.maximum(lmax, _shift_right(sm, s, -1.0))
        rmax = jnp.maximum(rmax, _shift_left(sm, s, -1.0))
    valid = peak & (sm > lmax) & (sm >= rmax)

    score = jnp.where(valid, sm, -1.0)
    iota = jax.lax.broadcasted_iota(jnp.int32, (_SPB, _NS), 1)
    freq_row = fq_ref[...][:, :_NS]

    pfs = []
    for j in range(_NPK):
        m = jnp.max(score, axis=1)
        idx = jnp.argmax(score, axis=1)
        hit = iota == idx[:, None]
        freq = jnp.sum(jnp.where(hit, freq_row, 0.0), axis=1)
        exists = m >= 0.0
        pfs.append(jnp.where(exists, freq, rf_ref[:, j]))
        score = jnp.where(hit, -1.0, score)

    p0, p1, p2 = pfs
    ab_lo = jnp.minimum(p0, p1)
    ab_hi = jnp.maximum(p0, p1)
    lo = jnp.minimum(ab_lo, p2)
    hi = jnp.maximum(ab_hi, p2)
    mid = jnp.maximum(ab_lo, jnp.minimum(ab_hi, p2))

    c1 = (mid - lo) / 2.0 + lo
    c2 = (hi - mid) / 2.0 + mid

    col = jax.lax.broadcasted_iota(jnp.int32, (_SPB, 128), 1)
    out = jnp.where(col == 0, jnp.float32(0.5),
          jnp.where(col <= 2, c1[:, None],
          jnp.where(col <= 4, c2[:, None],
          jnp.where(col == 5, jnp.float32(45.0), jnp.float32(0.0)))))
    anch_ref[...] = out


def kernel(x):
    x2 = x.reshape(-1, _L)
    r = x2.shape[0]
    b = r // _C

    cmat, smat = pl.pallas_call(
        _twiddle_body,
        out_specs=[pl.BlockSpec((_L, _NPAD), lambda: (0, 0)),
                   pl.BlockSpec((_L, _NPAD), lambda: (0, 0))],
        out_shape=[jax.ShapeDtypeStruct((_L, _NPAD), jnp.float32),
                   jax.ShapeDtypeStruct((_L, _NPAD), jnp.float32)],
    )()

    rkey = jax.random.fold_in(jax.random.key(8989), 1)
    rand_idx = jax.random.randint(rkey, (b, _NPK), 0, _NS)
    rand_freq = jnp.take(jnp.asarray(_FREQS), rand_idx, axis=0)
    rf = jnp.zeros((b, 128), dtype=jnp.float32).at[:, :_NPK].set(rand_freq)

    grid = (r // _ROWS,)
    fd, anch = pl.pallas_call(
        _body,
        grid=grid,
        in_specs=[
            pl.BlockSpec((_ROWS, _L), lambda i: (i, 0)),
            pl.BlockSpec((_L, _NPAD), lambda i: (0, 0)),
            pl.BlockSpec((_L, _NPAD), lambda i: (0, 0)),
            pl.BlockSpec((_SPB, 128), lambda i: (i, 0)),
            pl.BlockSpec((_SPB, _NPAD), lambda i: (0, 0)),
        ],
        out_specs=[
            pl.BlockSpec((_ROWS, _NF), lambda i: (i, 0)),
            pl.BlockSpec((_SPB, 128), lambda i: (i, 0)),
        ],
        out_shape=[
            jax.ShapeDtypeStruct((r, _NF), jnp.float32),
            jax.ShapeDtypeStruct((b, 128), jnp.float32),
        ],
    )(x2, cmat, smat, rf, jnp.asarray(_FREQ_ROW))

    anchors = anch[:, :6].reshape(b, 3, 2)
    f_ = fd.reshape(b, _C, _NF)
    return (anchors, f_)


# speedup vs baseline: 4.3333x; 1.1408x over previous
import ml_dtypes
import numpy as np
import jax
import jax.numpy as jnp
from jax.experimental import pallas as pl

_C = 32
_L = 500
_NF = _L // 2 + 1
_NS = _L // 2 - 1
_NPK = 3
_WIN = 11
_SPB = 32
_ROWS = _SPB * _C
_NPAD = 256

_OMEGA = float(np.float32(-2.0 * np.pi / _L))

_k3 = np.arange(-1, 2, dtype=np.float32)
_gk = np.exp(-_k3 ** 2 / (2.0 * np.float32(2.0) ** 2)).astype(np.float32)
_gk = (_gk / _gk.sum()).astype(np.float32)
_GA16 = float(np.float32(_gk[0].astype(ml_dtypes.bfloat16)))
_GB16 = float(np.float32(_gk[1].astype(ml_dtypes.bfloat16)))

_FREQS = (np.arange(_NF, dtype=np.float32)
          * np.float32(1.0 / (_L * (1.0 / 128.0))))[1:_NF - 1]
_FREQ_ROW = np.zeros((_SPB, _NPAD), dtype=np.float32)
_FREQ_ROW[:, :_NS] = _FREQS[None, :]


def _twiddle_body(c_ref, s_ref):
    j = jax.lax.broadcasted_iota(jnp.uint32, (_L, _NPAD), 0)
    k = jax.lax.broadcasted_iota(jnp.uint32, (_L, _NPAD), 1)
    theta = ((j * k) % jnp.uint32(_L)).astype(jnp.float32) * jnp.float32(_OMEGA)
    c = jnp.cos(theta)
    s = jnp.sin(theta)
    c_ref[...] = c
    s_ref[...] = s - c


def _shift_right(a, s, fill):
    pad = jnp.full(a.shape[:-1] + (s,), fill, dtype=a.dtype)
    return jnp.concatenate([pad, a[..., :-s]], axis=-1)


def _shift_left(a, s, fill):
    pad = jnp.full(a.shape[:-1] + (s,), fill, dtype=a.dtype)
    return jnp.concatenate([a[..., s:], pad], axis=-1)


def _body(x_ref, c_ref, s_ref, rf_ref, fq_ref, fd_ref, anch_ref):
    xb = x_ref[...]
    re = jax.lax.dot_general(
        xb, c_ref[...], (((1,), (0,)), ((), ())),
        preferred_element_type=jnp.float32,
        precision=jax.lax.Precision.HIGHEST)
    bm = jax.lax.dot_general(
        xb, s_ref[...], (((1,), (0,)), ((), ())),
        preferred_element_type=jnp.float32,
        precision=jax.lax.Precision.HIGHEST)
    im = re + bm
    fd = re * re + im * im
    fd_ref[...] = fd[:, :_NF]

    fd3 = fd.reshape(_SPB, _C, _NPAD)
    acc = fd3[:, 0, :]
    for c in range(1, _C):
        acc = acc + fd3[:, c, :]
    fm = acc * jnp.float32(0.03125)
    f = fm[:, 1:1 + _NS]

    fb = f.astype(jnp.bfloat16).astype(jnp.float32)
    fl = _shift_right(fb, 1, 0.0)
    fr = _shift_left(fb, 1, 0.0)
    sm = _GA16 * fl + _GB16 * fb + _GA16 * fr

    inf = jnp.float32(jnp.inf)
    sm_l = _shift_right(sm, 1, inf)
    sm_r = _shift_left(sm, 1, inf)
    peak = ((sm - sm_l) > 0) & ((sm_r - sm) < 0)

    lmax = jnp.full(sm.shape, -1.0, dtype=sm.dtype)
    rmax = jnp.full(sm.shape, -1.0, dtype=sm.dtype)
    for s in range(1, _WIN + 1):
        lmax = jnp.maximum(lmax, _shift_right(sm, s, -1.0))
        rmax = jnp.maximum(rmax, _shift_left(sm, s, -1.0))
    valid = peak & (sm > lmax) & (sm >= rmax)

    score = jnp.where(valid, sm, -1.0)
    iota = jax.lax.broadcasted_iota(jnp.int32, (_SPB, _NS), 1)
    freq_row = fq_ref[...][:, :_NS]

    pfs = []
    for j in range(_NPK):
        m = jnp.max(score, axis=1)
        idx = jnp.argmax(score, axis=1)
        hit = iota == idx[:, None]
        freq = jnp.sum(jnp.where(hit, freq_row, 0.0), axis=1)
        exists = m >= 0.0
        pfs.append(jnp.where(exists, freq, rf_ref[:, j]))
        score = jnp.where(hit, -1.0, score)

    p0, p1, p2 = pfs
    ab_lo = jnp.minimum(p0, p1)
    ab_hi = jnp.maximum(p0, p1)
    lo = jnp.minimum(ab_lo, p2)
    hi = jnp.maximum(ab_hi, p2)
    mid = jnp.maximum(ab_lo, jnp.minimum(ab_hi, p2))

    c1 = (mid - lo) / 2.0 + lo
    c2 = (hi - mid) / 2.0 + mid

    col = jax.lax.broadcasted_iota(jnp.int32, (_SPB, 128), 1)
    out = jnp.where(col == 0, jnp.float32(0.5),
          jnp.where(col <= 2, c1[:, None],
          jnp.where(col <= 4, c2[:, None],
          jnp.where(col == 5, jnp.float32(45.0), jnp.float32(0.0)))))
    anch_ref[...] = out


def kernel(x):
    x2 = x.reshape(-1, _L)
    r = x2.shape[0]
    b = r // _C

    cmat, smat = pl.pallas_call(
        _twiddle_body,
        out_specs=[pl.BlockSpec((_L, _NPAD), lambda: (0, 0)),
                   pl.BlockSpec((_L, _NPAD), lambda: (0, 0))],
        out_shape=[jax.ShapeDtypeStruct((_L, _NPAD), jnp.float32),
                   jax.ShapeDtypeStruct((_L, _NPAD), jnp.float32)],
    )()

    rkey = jax.random.fold_in(jax.random.key(8989), 1)
    rand_idx = jax.random.randint(rkey, (b, _NPK), 0, _NS)
    rand_freq = jnp.take(jnp.asarray(_FREQS), rand_idx, axis=0)
    rf = jnp.zeros((b, 128), dtype=jnp.float32).at[:, :_NPK].set(rand_freq)

    grid = (r // _ROWS,)
    fd, anch = pl.pallas_call(
        _body,
        grid=grid,
        in_specs=[
            pl.BlockSpec((_ROWS, _L), lambda i: (i, 0)),
            pl.BlockSpec((_L, _NPAD), lambda i: (0, 0)),
            pl.BlockSpec((_L, _NPAD), lambda i: (0, 0)),
            pl.BlockSpec((_SPB, 128), lambda i: (i, 0)),
            pl.BlockSpec((_SPB, _NPAD), lambda i: (0, 0)),
        ],
        out_specs=[
            pl.BlockSpec((_ROWS, _NF), lambda i: (i, 0)),
            pl.BlockSpec((_SPB, 128), lambda i: (i, 0)),
        ],
        out_shape=[
            jax.ShapeDtypeStruct((r, _NF), jnp.float32),
            jax.ShapeDtypeStruct((b, 128), jnp.float32),
        ],
    )(x2, cmat, smat, rf, jnp.asarray(_FREQ_ROW))

    anchors = anch[:, :6].reshape(b, 3, 2)
    f_ = fd.reshape(b, _C, _NF)
    return (anchors, f_)
